# baseline (speedup 1.0000x reference)
import functools

import jax
import jax.numpy as jnp
from jax import lax
from jax.experimental import pallas as pl
from jax.experimental.pallas import tpu as pltpu
from jax.experimental.pallas import tpu_sc as plsc

_N_NODES = 10000
_N_EDGES = 160000
_D1 = 256
_D2 = 128

_NPAD = 10240

_NC = 2
_NS = 16
_NW = _NC * _NS

_NB = 1000
_EB = 1000
_GCH = 80
_SCH = 40


def _relu(x):
    return jnp.maximum(x, 0.0)


def _dot(x, w):
    return jnp.dot(x.astype(jnp.bfloat16), w, preferred_element_type=jnp.float32)


def _node_pre(fin, wd1, bd1, wd2, bd2, wvsk, bvsk, wvrk, bvrk, wvi, bvi):
    f = fin.shape[1]
    grid = (_N_NODES // _NB,)

    def body(x_ref, wd1r, bd1r, wd2r, bd2r, wvskr, bvskr, wvrkr, bvrkr,
             wvir, bvir, hx_ref, t_ref, vi_ref):
        x = x_ref[...]
        h = _relu(_dot(x, wd1r[...]) + bd1r[...])
        hx = _relu(_dot(h, wd2r[...]) + bd2r[...])
        hx_ref[...] = hx
        t_ref[0] = _dot(hx, wvskr[...]) + bvskr[...]
        t_ref[1] = _dot(hx, wvrkr[...]) + bvrkr[...]
        vi_ref[...] = _dot(hx, wvir[...]) + bvir[...]

    full = lambda a: pl.BlockSpec(a.shape, lambda i: (0,) * a.ndim)
    return pl.pallas_call(
        body,
        grid=grid,
        in_specs=[pl.BlockSpec((_NB, f), lambda i: (i, 0))]
        + [full(a) for a in (wd1, bd1, wd2, bd2, wvsk, bvsk, wvrk, bvrk, wvi, bvi)],
        out_specs=[
            pl.BlockSpec((_NB, _D2), lambda i: (i, 0)),
            pl.BlockSpec((2, _NB, _D1), lambda i: (0, i, 0)),
            pl.BlockSpec((_NB, _D1), lambda i: (i, 0)),
        ],
        out_shape=[
            jax.ShapeDtypeStruct((_N_NODES, _D2), jnp.float32),
            jax.ShapeDtypeStruct((2, _N_NODES, _D1), jnp.float32),
            jax.ShapeDtypeStruct((_N_NODES, _D1), jnp.float32),
        ],
    )(fin, wd1, bd1, wd2, bd2, wvsk, bvsk, wvrk, bvrk, wvi, bvi)


def _edge(ein, gs, gd, wd1, bd1, wd2, bd2, wek, bek, wm1, bm1, wm2, bm2,
          residual_he):
    f = ein.shape[1]
    grid = (_N_EDGES // _EB,)

    def body(x_ref, gs_ref, gd_ref, wd1r, bd1r, wd2r, bd2r, wekr, bekr,
             wm1r, bm1r, wm2r, bm2r, e_ref, eo_ref):
        x = x_ref[...]
        h = _relu(_dot(x, wd1r[...]) + bd1r[...])
        he = _relu(_dot(h, wd2r[...]) + bd2r[...])
        ek = _dot(he, wekr[...]) + bekr[...]
        e0 = _relu(gs_ref[...] + gd_ref[...] + ek)
        h1 = _relu(_dot(e0, wm1r[...]) + bm1r[...])
        e = _relu(_dot(h1, wm2r[...]) + bm2r[...])
        e_ref[...] = e
        eo_ref[...] = e + (he if residual_he else x)

    full = lambda a: pl.BlockSpec(a.shape, lambda i: (0,) * a.ndim)
    return pl.pallas_call(
        body,
        grid=grid,
        in_specs=[
            pl.BlockSpec((_EB, f), lambda i: (i, 0)),
            pl.BlockSpec((_EB, _D1), lambda i: (i, 0)),
            pl.BlockSpec((_EB, _D1), lambda i: (i, 0)),
        ]
        + [full(a) for a in (wd1, bd1, wd2, bd2, wek, bek, wm1, bm1, wm2, bm2)],
        out_specs=[
            pl.BlockSpec((_EB, _D2), lambda i: (i, 0)),
            pl.BlockSpec((_EB, _D2), lambda i: (i, 0)),
        ],
        out_shape=[
            jax.ShapeDtypeStruct((_N_EDGES, _D2), jnp.float32),
            jax.ShapeDtypeStruct((_N_EDGES, _D2), jnp.float32),
        ],
    )(ein, gs, gd, wd1, bd1, wd2, bd2, wek, bek, wm1, bm1, wm2, bm2)


def _node_upd(parts, cnts, vi, res, wvie, bvie, wn1, bn1, wn2, bn2):
    grid = (_N_NODES // _NB,)

    def body(p_ref, c_ref, vi_ref, r_ref, wvier, bvier, wn1r, bn1r, wn2r,
             bn2r, vo_ref):
        agg = p_ref[0] + p_ref[1]
        cnt = (c_ref[0] + c_ref[1])[:, 0:1]
        agg = agg / jnp.maximum(cnt, 1.0)
        vie = _dot(agg, wvier[...]) + bvier[...]
        v = _relu(vi_ref[...] + vie)
        v = _relu(_dot(v, wn1r[...]) + bn1r[...])
        v = _relu(_dot(v, wn2r[...]) + bn2r[...])
        vo_ref[...] = v + r_ref[...]

    full = lambda a: pl.BlockSpec(a.shape, lambda i: (0,) * a.ndim)
    return pl.pallas_call(
        body,
        grid=grid,
        in_specs=[
            pl.BlockSpec((2, _NB, _D2), lambda i: (0, i, 0)),
            pl.BlockSpec((2, _NB, _D2), lambda i: (0, i, 0)),
            pl.BlockSpec((_NB, _D1), lambda i: (i, 0)),
            pl.BlockSpec((_NB, _D2), lambda i: (i, 0)),
        ]
        + [full(a) for a in (wvie, bvie, wn1, bn1, wn2, bn2)],
        out_specs=pl.BlockSpec((_NB, _D2), lambda i: (i, 0)),
        out_shape=jax.ShapeDtypeStruct((_N_NODES, _D2), jnp.float32),
    )(parts, cnts, vi, res, wvie, bvie, wn1, bn1, wn2, bn2)


def _sc_gather(table, idx3):
    n_ch = idx3.shape[1]
    b = _NW * n_ch * _GCH
    d = table.shape[1]
    per_w = b // _NW
    mesh = plsc.VectorSubcoreMesh(core_axis_name="c", subcore_axis_name="s")
    n_pair = (n_ch - 1) // 2

    @functools.partial(
        pl.kernel,
        mesh=mesh,
        out_type=jax.ShapeDtypeStruct((b, d), jnp.float32),
        scratch_types=[
            pltpu.VMEM((n_ch, _GCH), jnp.int32),
            pltpu.VMEM((_GCH, d), jnp.float32),
            pltpu.VMEM((_GCH, d), jnp.float32),
            pltpu.SemaphoreType.DMA,
            pltpu.SemaphoreType.DMA,
        ],
    )
    def k(table_hbm, idx3_hbm, out_hbm, idx_v, rows_a, rows_b, sem_a, sem_b):
        wid = lax.axis_index("s") * _NC + lax.axis_index("c")
        base = wid * per_w
        pltpu.sync_copy(idx3_hbm.at[wid], idx_v)
        pltpu.async_copy(table_hbm.at[idx_v.at[0]], rows_a, sem_a)

        def wait_gather(rows, sem):
            pltpu.make_async_copy(table_hbm.at[idx_v.at[0]], rows, sem).wait()

        def step(g, carry):
            c0 = 2 * g
            pltpu.async_copy(table_hbm.at[idx_v.at[c0 + 1]], rows_b, sem_b)
            wait_gather(rows_a, sem_a)
            pltpu.sync_copy(rows_a, out_hbm.at[pl.ds(base + c0 * _GCH, _GCH)])
            pltpu.async_copy(table_hbm.at[idx_v.at[c0 + 2]], rows_a, sem_a)
            wait_gather(rows_b, sem_b)
            pltpu.sync_copy(rows_b,
                            out_hbm.at[pl.ds(base + (c0 + 1) * _GCH, _GCH)])
            return carry

        lax.fori_loop(0, n_pair, step, 0)
        wait_gather(rows_a, sem_a)
        pltpu.sync_copy(rows_a,
                        out_hbm.at[pl.ds(base + (n_ch - 1) * _GCH, _GCH)])

    return k(table, idx3)


def _sc_scatter(vals, idx3, zeros_acc):
    e = vals.shape[0]
    d = vals.shape[1]
    per_w = e // _NW
    n_ch = per_w // _SCH
    rows_t = _NPAD // _NS
    drc = rows_t // 4
    mesh = plsc.VectorSubcoreMesh(core_axis_name="c", subcore_axis_name="s")

    @functools.partial(
        pl.kernel,
        mesh=mesh,
        out_type=jax.ShapeDtypeStruct((_NC, _NPAD, d), jnp.float32),
        scratch_types=[
            pltpu.VMEM((n_ch, _SCH), jnp.int32),
            pltpu.VMEM((_SCH, d), jnp.float32),
            pltpu.VMEM((_SCH, d), jnp.float32),
            pltpu.VMEM((drc, d), jnp.float32),
            pltpu.VMEM_SHARED((_NPAD, d), jnp.float32),
            pltpu.SemaphoreType.DMA,
            pltpu.SemaphoreType.DMA,
        ],
    )
    def k(vals_hbm, idx3_hbm, zacc_hbm, parts_hbm, idx_v, rows_a, rows_b,
          buf_v, acc_sh, sem_a, sem_b):
        c = lax.axis_index("c")
        s = lax.axis_index("s")
        wid = s * _NC + c
        r0 = s * rows_t
        pltpu.sync_copy(zacc_hbm, buf_v)
        for h in range(4):
            pltpu.sync_copy(buf_v, acc_sh.at[pl.ds(r0 + h * drc, drc)])
        pltpu.sync_copy(idx3_hbm.at[wid], idx_v)
        plsc.subcore_barrier()
        base = wid * per_w
        n_pair = (n_ch - 1) // 2

        def load(i, rows, sem):
            pltpu.async_copy(vals_hbm.at[pl.ds(base + i * _SCH, _SCH)],
                             rows, sem)

        def wait_load(i, rows, sem):
            pltpu.make_async_copy(
                vals_hbm.at[pl.ds(base + i * _SCH, _SCH)], rows, sem).wait()

        load(0, rows_a, sem_a)

        def step(g, carry):
            c0 = 2 * g
            load(c0 + 1, rows_b, sem_b)
            wait_load(c0, rows_a, sem_a)
            pltpu.sync_copy(rows_a, acc_sh.at[idx_v.at[c0]], add=True)
            load(c0 + 2, rows_a, sem_a)
            wait_load(c0 + 1, rows_b, sem_b)
            pltpu.sync_copy(rows_b, acc_sh.at[idx_v.at[c0 + 1]], add=True)
            return carry

        lax.fori_loop(0, n_pair, step, 0)
        wait_load(n_ch - 1, rows_a, sem_a)
        pltpu.sync_copy(rows_a, acc_sh.at[idx_v.at[n_ch - 1]], add=True)
        plsc.subcore_barrier()
        for h in range(4):
            pltpu.sync_copy(acc_sh.at[pl.ds(r0 + h * drc, drc)], buf_v)
            pltpu.sync_copy(buf_v, parts_hbm.at[c, pl.ds(r0 + h * drc, drc)])

    return k(vals, idx3, zeros_acc)


def _tw(lp):
    return lp["w"].T.astype(jnp.bfloat16), lp["b"][None, :]


def kernel(feat, efeat, edge_index, params):
    src = edge_index[0].astype(jnp.int32)
    dst = edge_index[1].astype(jnp.int32)
    idx_all = jnp.concatenate([src, dst + _N_NODES])
    idx_all3 = idx_all.reshape(_NW, (2 * _N_EDGES // _NW) // _GCH, _GCH)
    dst3 = dst.reshape(_NW, (_N_EDGES // _NW) // _SCH, _SCH)
    ones_e = jnp.ones((_N_EDGES, _D2), jnp.float32)
    zeros_acc = jnp.zeros((_NPAD // _NS // 4, _D2), jnp.float32)

    cnts = None
    vout, eout = feat, efeat
    for n, bp in enumerate(params["blocks"]):
        fin, ein = vout, eout
        wd1, bd1 = _tw(bp["dense_node"][0])
        wd2, bd2 = _tw(bp["dense_node"][1])
        we1, be1 = _tw(bp["dense_edge"][0])
        we2, be2 = _tw(bp["dense_edge"][1])
        wvsk, bvsk = _tw(bp["edge_mlp0"]["vsk"])
        wvrk, bvrk = _tw(bp["edge_mlp0"]["vrk"])
        wek, bek = _tw(bp["edge_mlp0"]["ek"])
        wm1, bm1 = _tw(bp["edge_mlp"][0])
        wm2, bm2 = _tw(bp["edge_mlp"][1])
        wvi, bvi = _tw(bp["node_mlp0"]["vi"])
        wvie, bvie = _tw(bp["node_mlp0"]["vie"])
        wn1, bn1 = _tw(bp["node_mlp"][0])
        wn2, bn2 = _tw(bp["node_mlp"][1])

        hx, t, vi = _node_pre(fin, wd1, bd1, wd2, bd2, wvsk, bvsk,
                              wvrk, bvrk, wvi, bvi)
        g = _sc_gather(t.reshape(2 * _N_NODES, _D1), idx_all3)
        if n == 0:
            cnts = _sc_scatter(ones_e, dst3, zeros_acc)
        e, eo = _edge(ein, g[:_N_EDGES], g[_N_EDGES:], we1, be1, we2, be2,
                      wek, bek, wm1, bm1, wm2, bm2, residual_he=(n == 0))
        parts = _sc_scatter(e, dst3, zeros_acc)
        res = hx if n == 0 else fin
        v = _node_upd(parts[:, :_N_NODES], cnts[:, :_N_NODES], vi, res,
                      wvie, bvie, wn1, bn1, wn2, bn2)
        vout, eout = v, eo
    return vout, eout


# speedup vs baseline: 1.2768x; 1.2768x over previous
import functools

import jax
import jax.numpy as jnp
from jax import lax
from jax.experimental import pallas as pl
from jax.experimental.pallas import tpu as pltpu
from jax.experimental.pallas import tpu_sc as plsc

_N_NODES = 10000
_N_EDGES = 160000
_D1 = 256
_D2 = 128

_NPAD = 10240

_NC = 2
_NS = 16
_NW = _NC * _NS

_NB = 1000
_EB = 1000
_GCH = 80
_SCH = 40


def _relu(x):
    return jnp.maximum(x, 0.0)


def _dot(x, w):
    return jnp.dot(x.astype(jnp.bfloat16), w, preferred_element_type=jnp.float32)


def _node_pre(fin, wd1, bd1, wd2, bd2, wvi, bvi):
    f = fin.shape[1]
    grid = (_N_NODES // _NB,)

    def body(x_ref, wd1r, bd1r, wd2r, bd2r, wvir, bvir, hx_ref, vi_ref):
        x = x_ref[...]
        h = _relu(_dot(x, wd1r[...]) + bd1r[...])
        hx = _relu(_dot(h, wd2r[...]) + bd2r[...])
        hx_ref[...] = hx
        vi_ref[...] = _dot(hx, wvir[...]) + bvir[...]

    full = lambda a: pl.BlockSpec(a.shape, lambda i: (0,) * a.ndim)
    return pl.pallas_call(
        body,
        grid=grid,
        in_specs=[pl.BlockSpec((_NB, f), lambda i: (i, 0))]
        + [full(a) for a in (wd1, bd1, wd2, bd2, wvi, bvi)],
        out_specs=[
            pl.BlockSpec((_NB, _D2), lambda i: (i, 0)),
            pl.BlockSpec((_NB, _D1), lambda i: (i, 0)),
        ],
        out_shape=[
            jax.ShapeDtypeStruct((_N_NODES, _D2), jnp.float32),
            jax.ShapeDtypeStruct((_N_NODES, _D1), jnp.float32),
        ],
    )(fin, wd1, bd1, wd2, bd2, wvi, bvi)


def _edge(ein, gs, gd, wd1, bd1, wd2, bd2, wvsk, wvrk, wek, b0, wm1, bm1,
          wm2, bm2, residual_he):
    f = ein.shape[1]
    grid = (_N_EDGES // _EB,)

    def body(x_ref, gs_ref, gd_ref, wd1r, bd1r, wd2r, bd2r, wvskr, wvrkr,
             wekr, b0r, wm1r, bm1r, wm2r, bm2r, e_ref, eo_ref):
        x = x_ref[...]
        h = _relu(_dot(x, wd1r[...]) + bd1r[...])
        he = _relu(_dot(h, wd2r[...]) + bd2r[...])
        e0 = _relu(_dot(gs_ref[...], wvskr[...]) + _dot(gd_ref[...], wvrkr[...])
                   + _dot(he, wekr[...]) + b0r[...])
        h1 = _relu(_dot(e0, wm1r[...]) + bm1r[...])
        e = _relu(_dot(h1, wm2r[...]) + bm2r[...])
        e_ref[...] = e
        eo_ref[...] = e + (he if residual_he else x)

    full = lambda a: pl.BlockSpec(a.shape, lambda i: (0,) * a.ndim)
    return pl.pallas_call(
        body,
        grid=grid,
        in_specs=[
            pl.BlockSpec((_EB, f), lambda i: (i, 0)),
            pl.BlockSpec((_EB, _D2), lambda i: (i, 0)),
            pl.BlockSpec((_EB, _D2), lambda i: (i, 0)),
        ]
        + [full(a) for a in (wd1, bd1, wd2, bd2, wvsk, wvrk, wek, b0, wm1,
                             bm1, wm2, bm2)],
        out_specs=[
            pl.BlockSpec((_EB, _D2), lambda i: (i, 0)),
            pl.BlockSpec((_EB, _D2), lambda i: (i, 0)),
        ],
        out_shape=[
            jax.ShapeDtypeStruct((_N_EDGES, _D2), jnp.float32),
            jax.ShapeDtypeStruct((_N_EDGES, _D2), jnp.float32),
        ],
    )(ein, gs, gd, wd1, bd1, wd2, bd2, wvsk, wvrk, wek, b0, wm1, bm1,
      wm2, bm2)


def _node_upd(parts, cnts, vi, res, wvie, bvie, wn1, bn1, wn2, bn2):
    grid = (_N_NODES // _NB,)

    def body(p_ref, c_ref, vi_ref, r_ref, wvier, bvier, wn1r, bn1r, wn2r,
             bn2r, vo_ref):
        agg = p_ref[0] + p_ref[1]
        cnt = (c_ref[0] + c_ref[1])[:, 0:1]
        agg = agg / jnp.maximum(cnt, 1.0)
        vie = _dot(agg, wvier[...]) + bvier[...]
        v = _relu(vi_ref[...] + vie)
        v = _relu(_dot(v, wn1r[...]) + bn1r[...])
        v = _relu(_dot(v, wn2r[...]) + bn2r[...])
        vo_ref[...] = v + r_ref[...]

    full = lambda a: pl.BlockSpec(a.shape, lambda i: (0,) * a.ndim)
    return pl.pallas_call(
        body,
        grid=grid,
        in_specs=[
            pl.BlockSpec((2, _NB, _D2), lambda i: (0, i, 0)),
            pl.BlockSpec((2, _NB, _D2), lambda i: (0, i, 0)),
            pl.BlockSpec((_NB, _D1), lambda i: (i, 0)),
            pl.BlockSpec((_NB, _D2), lambda i: (i, 0)),
        ]
        + [full(a) for a in (wvie, bvie, wn1, bn1, wn2, bn2)],
        out_specs=pl.BlockSpec((_NB, _D2), lambda i: (i, 0)),
        out_shape=jax.ShapeDtypeStruct((_N_NODES, _D2), jnp.float32),
    )(parts, cnts, vi, res, wvie, bvie, wn1, bn1, wn2, bn2)


def _sc_gather(table, idx3):
    n_ch = idx3.shape[1]
    b = _NW * n_ch * _GCH
    d = table.shape[1]
    per_w = b // _NW
    mesh = plsc.VectorSubcoreMesh(core_axis_name="c", subcore_axis_name="s")
    n_pair = (n_ch - 1) // 2

    @functools.partial(
        pl.kernel,
        mesh=mesh,
        out_type=jax.ShapeDtypeStruct((b, d), jnp.float32),
        scratch_types=[
            pltpu.VMEM((n_ch, _GCH), jnp.int32),
            pltpu.VMEM((_GCH, d), jnp.float32),
            pltpu.VMEM((_GCH, d), jnp.float32),
            pltpu.SemaphoreType.DMA,
            pltpu.SemaphoreType.DMA,
        ],
    )
    def k(table_hbm, idx3_hbm, out_hbm, idx_v, rows_a, rows_b, sem_a, sem_b):
        wid = lax.axis_index("s") * _NC + lax.axis_index("c")
        base = wid * per_w
        pltpu.sync_copy(idx3_hbm.at[wid], idx_v)
        pltpu.async_copy(table_hbm.at[idx_v.at[0]], rows_a, sem_a)

        def wait_gather(rows, sem):
            pltpu.make_async_copy(table_hbm.at[idx_v.at[0]], rows, sem).wait()

        def step(g, carry):
            c0 = 2 * g
            pltpu.async_copy(table_hbm.at[idx_v.at[c0 + 1]], rows_b, sem_b)
            wait_gather(rows_a, sem_a)
            pltpu.sync_copy(rows_a, out_hbm.at[pl.ds(base + c0 * _GCH, _GCH)])
            pltpu.async_copy(table_hbm.at[idx_v.at[c0 + 2]], rows_a, sem_a)
            wait_gather(rows_b, sem_b)
            pltpu.sync_copy(rows_b,
                            out_hbm.at[pl.ds(base + (c0 + 1) * _GCH, _GCH)])
            return carry

        lax.fori_loop(0, n_pair, step, 0)
        wait_gather(rows_a, sem_a)
        pltpu.sync_copy(rows_a,
                        out_hbm.at[pl.ds(base + (n_ch - 1) * _GCH, _GCH)])

    return k(table, idx3)


def _sc_scatter(vals, idx3, zeros_acc):
    e = vals.shape[0]
    d = vals.shape[1]
    per_w = e // _NW
    n_ch = per_w // _SCH
    rows_t = _NPAD // _NS
    drc = rows_t // 4
    mesh = plsc.VectorSubcoreMesh(core_axis_name="c", subcore_axis_name="s")

    @functools.partial(
        pl.kernel,
        mesh=mesh,
        out_type=jax.ShapeDtypeStruct((_NC, _NPAD, d), jnp.float32),
        scratch_types=[
            pltpu.VMEM((n_ch, _SCH), jnp.int32),
            pltpu.VMEM((_SCH, d), jnp.float32),
            pltpu.VMEM((_SCH, d), jnp.float32),
            pltpu.VMEM((drc, d), jnp.float32),
            pltpu.VMEM_SHARED((_NPAD, d), jnp.float32),
            pltpu.SemaphoreType.DMA,
            pltpu.SemaphoreType.DMA,
        ],
    )
    def k(vals_hbm, idx3_hbm, zacc_hbm, parts_hbm, idx_v, rows_a, rows_b,
          buf_v, acc_sh, sem_a, sem_b):
        c = lax.axis_index("c")
        s = lax.axis_index("s")
        wid = s * _NC + c
        r0 = s * rows_t
        pltpu.sync_copy(zacc_hbm, buf_v)
        for h in range(4):
            pltpu.sync_copy(buf_v, acc_sh.at[pl.ds(r0 + h * drc, drc)])
        pltpu.sync_copy(idx3_hbm.at[wid], idx_v)
        plsc.subcore_barrier()
        base = wid * per_w
        n_pair = (n_ch - 1) // 2

        def load(i, rows, sem):
            pltpu.async_copy(vals_hbm.at[pl.ds(base + i * _SCH, _SCH)],
                             rows, sem)

        def wait_load(i, rows, sem):
            pltpu.make_async_copy(
                vals_hbm.at[pl.ds(base + i * _SCH, _SCH)], rows, sem).wait()

        load(0, rows_a, sem_a)

        def step(g, carry):
            c0 = 2 * g
            load(c0 + 1, rows_b, sem_b)
            wait_load(c0, rows_a, sem_a)
            pltpu.sync_copy(rows_a, acc_sh.at[idx_v.at[c0]], add=True)
            load(c0 + 2, rows_a, sem_a)
            wait_load(c0 + 1, rows_b, sem_b)
            pltpu.sync_copy(rows_b, acc_sh.at[idx_v.at[c0 + 1]], add=True)
            return carry

        lax.fori_loop(0, n_pair, step, 0)
        wait_load(n_ch - 1, rows_a, sem_a)
        pltpu.sync_copy(rows_a, acc_sh.at[idx_v.at[n_ch - 1]], add=True)
        plsc.subcore_barrier()
        for h in range(4):
            pltpu.sync_copy(acc_sh.at[pl.ds(r0 + h * drc, drc)], buf_v)
            pltpu.sync_copy(buf_v, parts_hbm.at[c, pl.ds(r0 + h * drc, drc)])

    return k(vals, idx3, zeros_acc)


def _tw(lp):
    return lp["w"].T.astype(jnp.bfloat16), lp["b"][None, :]


def kernel(feat, efeat, edge_index, params):
    src = edge_index[0].astype(jnp.int32)
    dst = edge_index[1].astype(jnp.int32)
    idx_all = jnp.concatenate([src, dst])
    idx_all3 = idx_all.reshape(_NW, (2 * _N_EDGES // _NW) // _GCH, _GCH)
    dst3 = dst.reshape(_NW, (_N_EDGES // _NW) // _SCH, _SCH)
    ones_e = jnp.ones((_N_EDGES, _D2), jnp.float32)
    zeros_acc = jnp.zeros((_NPAD // _NS // 4, _D2), jnp.float32)

    cnts = None
    vout, eout = feat, efeat
    for n, bp in enumerate(params["blocks"]):
        fin, ein = vout, eout
        wd1, bd1 = _tw(bp["dense_node"][0])
        wd2, bd2 = _tw(bp["dense_node"][1])
        we1, be1 = _tw(bp["dense_edge"][0])
        we2, be2 = _tw(bp["dense_edge"][1])
        wvsk, bvsk = _tw(bp["edge_mlp0"]["vsk"])
        wvrk, bvrk = _tw(bp["edge_mlp0"]["vrk"])
        wek, bek = _tw(bp["edge_mlp0"]["ek"])
        wm1, bm1 = _tw(bp["edge_mlp"][0])
        wm2, bm2 = _tw(bp["edge_mlp"][1])
        wvi, bvi = _tw(bp["node_mlp0"]["vi"])
        wvie, bvie = _tw(bp["node_mlp0"]["vie"])
        wn1, bn1 = _tw(bp["node_mlp"][0])
        wn2, bn2 = _tw(bp["node_mlp"][1])

        hx, vi = _node_pre(fin, wd1, bd1, wd2, bd2, wvi, bvi)
        g = _sc_gather(hx, idx_all3)
        if n == 0:
            cnts = _sc_scatter(ones_e, dst3, zeros_acc)
        b0 = bvsk + bvrk + bek
        e, eo = _edge(ein, g[:_N_EDGES], g[_N_EDGES:], we1, be1, we2, be2,
                      wvsk, wvrk, wek, b0, wm1, bm1, wm2, bm2,
                      residual_he=(n == 0))
        parts = _sc_scatter(e, dst3, zeros_acc)
        res = hx if n == 0 else fin
        v = _node_upd(parts[:, :_N_NODES], cnts[:, :_N_NODES], vi, res,
                      wvie, bvie, wn1, bn1, wn2, bn2)
        vout, eout = v, eo
    return vout, eout


# speedup vs baseline: 1.3183x; 1.0325x over previous
import functools

import jax
import jax.numpy as jnp
from jax import lax
from jax.experimental import pallas as pl
from jax.experimental.pallas import tpu as pltpu
from jax.experimental.pallas import tpu_sc as plsc

_N_NODES = 10000
_N_EDGES = 160000
_D1 = 256
_D2 = 128

_NPAD = 10240

_NC = 2
_NS = 16
_NW = _NC * _NS

_NB = 1000
_EB = 1000
_GCH = 80
_SCH = 40


def _relu(x):
    return jnp.maximum(x, 0.0)


def _dot(x, w):
    return jnp.dot(x.astype(jnp.bfloat16), w, preferred_element_type=jnp.float32)


def _node_pre(fin, wd1, bd1, wd2, bd2, wvi, bvi):
    f = fin.shape[1]
    grid = (_N_NODES // _NB,)

    def body(x_ref, wd1r, bd1r, wd2r, bd2r, wvir, bvir, hx_ref, vi_ref):
        x = x_ref[...]
        h = _relu(_dot(x, wd1r[...]) + bd1r[...])
        hx = _relu(_dot(h, wd2r[...]) + bd2r[...])
        hx_ref[...] = hx
        vi_ref[...] = _dot(hx, wvir[...]) + bvir[...]

    full = lambda a: pl.BlockSpec(a.shape, lambda i: (0,) * a.ndim)
    return pl.pallas_call(
        body,
        grid=grid,
        in_specs=[pl.BlockSpec((_NB, f), lambda i: (i, 0))]
        + [full(a) for a in (wd1, bd1, wd2, bd2, wvi, bvi)],
        out_specs=[
            pl.BlockSpec((_NB, _D2), lambda i: (i, 0)),
            pl.BlockSpec((_NB, _D1), lambda i: (i, 0)),
        ],
        out_shape=[
            jax.ShapeDtypeStruct((_N_NODES, _D2), jnp.float32),
            jax.ShapeDtypeStruct((_N_NODES, _D1), jnp.float32),
        ],
    )(fin, wd1, bd1, wd2, bd2, wvi, bvi)


def _edge(ein, gs, gd, wd1, bd1, wd2, bd2, wvsk, wvrk, wek, b0, wm1, bm1,
          wm2, bm2, residual_he):
    f = ein.shape[1]
    grid = (_N_EDGES // _EB,)

    def body(x_ref, gs_ref, gd_ref, wd1r, bd1r, wd2r, bd2r, wvskr, wvrkr,
             wekr, b0r, wm1r, bm1r, wm2r, bm2r, e_ref, eo_ref):
        x = x_ref[...]
        h = _relu(_dot(x, wd1r[...]) + bd1r[...])
        he = _relu(_dot(h, wd2r[...]) + bd2r[...])
        e0 = _relu(_dot(gs_ref[...], wvskr[...]) + _dot(gd_ref[...], wvrkr[...])
                   + _dot(he, wekr[...]) + b0r[...])
        h1 = _relu(_dot(e0, wm1r[...]) + bm1r[...])
        e = _relu(_dot(h1, wm2r[...]) + bm2r[...])
        e_ref[...] = e
        eo_ref[...] = e + (he if residual_he else x)

    full = lambda a: pl.BlockSpec(a.shape, lambda i: (0,) * a.ndim)
    return pl.pallas_call(
        body,
        grid=grid,
        in_specs=[
            pl.BlockSpec((_EB, f), lambda i: (i, 0)),
            pl.BlockSpec((_EB, _D2), lambda i: (i, 0)),
            pl.BlockSpec((_EB, _D2), lambda i: (i, 0)),
        ]
        + [full(a) for a in (wd1, bd1, wd2, bd2, wvsk, wvrk, wek, b0, wm1,
                             bm1, wm2, bm2)],
        out_specs=[
            pl.BlockSpec((_EB, _D2), lambda i: (i, 0)),
            pl.BlockSpec((_EB, _D2), lambda i: (i, 0)),
        ],
        out_shape=[
            jax.ShapeDtypeStruct((_N_EDGES, _D2), jnp.float32),
            jax.ShapeDtypeStruct((_N_EDGES, _D2), jnp.float32),
        ],
    )(ein, gs, gd, wd1, bd1, wd2, bd2, wvsk, wvrk, wek, b0, wm1, bm1,
      wm2, bm2)


def _node_upd(parts, cnts, vi, res, wvie, bvie, wn1, bn1, wn2, bn2):
    grid = (_N_NODES // _NB,)

    def body(p_ref, c_ref, vi_ref, r_ref, wvier, bvier, wn1r, bn1r, wn2r,
             bn2r, vo_ref):
        agg = p_ref[0] + p_ref[1]
        cnt = (c_ref[0] + c_ref[1])[:, 0:1]
        agg = agg / jnp.maximum(cnt, 1.0)
        vie = _dot(agg, wvier[...]) + bvier[...]
        v = _relu(vi_ref[...] + vie)
        v = _relu(_dot(v, wn1r[...]) + bn1r[...])
        v = _relu(_dot(v, wn2r[...]) + bn2r[...])
        vo_ref[...] = v + r_ref[...]

    full = lambda a: pl.BlockSpec(a.shape, lambda i: (0,) * a.ndim)
    return pl.pallas_call(
        body,
        grid=grid,
        in_specs=[
            pl.BlockSpec((2, _NB, _D2), lambda i: (0, i, 0)),
            pl.BlockSpec((2, _NB, _D2), lambda i: (0, i, 0)),
            pl.BlockSpec((_NB, _D1), lambda i: (i, 0)),
            pl.BlockSpec((_NB, _D2), lambda i: (i, 0)),
        ]
        + [full(a) for a in (wvie, bvie, wn1, bn1, wn2, bn2)],
        out_specs=pl.BlockSpec((_NB, _D2), lambda i: (i, 0)),
        out_shape=jax.ShapeDtypeStruct((_N_NODES, _D2), jnp.float32),
    )(parts, cnts, vi, res, wvie, bvie, wn1, bn1, wn2, bn2)


def _sc_gather(table, idx3):
    n_ch = idx3.shape[1]
    b = _NW * n_ch * _GCH
    d = table.shape[1]
    per_w = b // _NW
    mesh = plsc.VectorSubcoreMesh(core_axis_name="c", subcore_axis_name="s")
    n_pair = (n_ch - 1) // 2

    @functools.partial(
        pl.kernel,
        mesh=mesh,
        out_type=jax.ShapeDtypeStruct((b, d), jnp.float32),
        scratch_types=[
            pltpu.VMEM((n_ch, _GCH), jnp.int32),
            pltpu.VMEM((_GCH, d), jnp.float32),
            pltpu.VMEM((_GCH, d), jnp.float32),
            pltpu.SemaphoreType.DMA,
            pltpu.SemaphoreType.DMA,
        ],
    )
    def k(table_hbm, idx3_hbm, out_hbm, idx_v, rows_a, rows_b, sem_a, sem_b):
        wid = lax.axis_index("s") * _NC + lax.axis_index("c")
        base = wid * per_w
        pltpu.sync_copy(idx3_hbm.at[wid], idx_v)
        pltpu.async_copy(table_hbm.at[idx_v.at[0]], rows_a, sem_a)

        def wait_gather(rows, sem):
            pltpu.make_async_copy(table_hbm.at[idx_v.at[0]], rows, sem).wait()

        def step(g, carry):
            c0 = 2 * g
            pltpu.async_copy(table_hbm.at[idx_v.at[c0 + 1]], rows_b, sem_b)
            wait_gather(rows_a, sem_a)
            pltpu.sync_copy(rows_a, out_hbm.at[pl.ds(base + c0 * _GCH, _GCH)])
            pltpu.async_copy(table_hbm.at[idx_v.at[c0 + 2]], rows_a, sem_a)
            wait_gather(rows_b, sem_b)
            pltpu.sync_copy(rows_b,
                            out_hbm.at[pl.ds(base + (c0 + 1) * _GCH, _GCH)])
            return carry

        lax.fori_loop(0, n_pair, step, 0)
        wait_gather(rows_a, sem_a)
        pltpu.sync_copy(rows_a,
                        out_hbm.at[pl.ds(base + (n_ch - 1) * _GCH, _GCH)])

    return k(table, idx3)


def _sc_scatter(vals, idx3, zeros_acc):
    e = vals.shape[0]
    d = vals.shape[1]
    per_w = e // _NW
    n_ch = per_w // _SCH
    rows_t = _NPAD // _NS
    drc = rows_t // 4
    mesh = plsc.VectorSubcoreMesh(core_axis_name="c", subcore_axis_name="s")

    @functools.partial(
        pl.kernel,
        mesh=mesh,
        out_type=jax.ShapeDtypeStruct((_NC, _NPAD, d), jnp.float32),
        scratch_types=[
            pltpu.VMEM((n_ch, _SCH), jnp.int32),
            pltpu.VMEM((_SCH, d), jnp.float32),
            pltpu.VMEM((_SCH, d), jnp.float32),
            pltpu.VMEM((drc, d), jnp.float32),
            pltpu.VMEM_SHARED((_NPAD, d), jnp.float32),
            pltpu.SemaphoreType.DMA,
            pltpu.SemaphoreType.DMA,
        ],
    )
    def k(vals_hbm, idx3_hbm, zacc_hbm, parts_hbm, idx_v, rows_a, rows_b,
          buf_v, acc_sh, sem_a, sem_b):
        c = lax.axis_index("c")
        s = lax.axis_index("s")
        wid = s * _NC + c
        r0 = s * rows_t
        pltpu.sync_copy(zacc_hbm, buf_v)
        for h in range(4):
            pltpu.sync_copy(buf_v, acc_sh.at[pl.ds(r0 + h * drc, drc)])
        pltpu.sync_copy(idx3_hbm.at[wid], idx_v)
        plsc.subcore_barrier()
        base = wid * per_w
        n_pair = (n_ch - 1) // 2

        def load(i, rows, sem):
            pltpu.async_copy(vals_hbm.at[pl.ds(base + i * _SCH, _SCH)],
                             rows, sem)

        def wait_load(i, rows, sem):
            pltpu.make_async_copy(
                vals_hbm.at[pl.ds(base + i * _SCH, _SCH)], rows, sem).wait()

        load(0, rows_a, sem_a)

        def step(g, carry):
            c0 = 2 * g
            load(c0 + 1, rows_b, sem_b)
            wait_load(c0, rows_a, sem_a)
            pltpu.sync_copy(rows_a, acc_sh.at[idx_v.at[c0]], add=True)
            load(c0 + 2, rows_a, sem_a)
            wait_load(c0 + 1, rows_b, sem_b)
            pltpu.sync_copy(rows_b, acc_sh.at[idx_v.at[c0 + 1]], add=True)
            return carry

        lax.fori_loop(0, n_pair, step, 0)
        wait_load(n_ch - 1, rows_a, sem_a)
        pltpu.sync_copy(rows_a, acc_sh.at[idx_v.at[n_ch - 1]], add=True)
        plsc.subcore_barrier()
        for h in range(4):
            pltpu.sync_copy(acc_sh.at[pl.ds(r0 + h * drc, drc)], buf_v)
            pltpu.sync_copy(buf_v, parts_hbm.at[c, pl.ds(r0 + h * drc, drc)])

    return k(vals, idx3, zeros_acc)


def _sc_count(idx3, zeros_acc, ones_rows):
    n_ch = idx3.shape[1]
    rows_t = _NPAD // _NS
    drc = rows_t // 4
    mesh = plsc.VectorSubcoreMesh(core_axis_name="c", subcore_axis_name="s")

    @functools.partial(
        pl.kernel,
        mesh=mesh,
        out_type=jax.ShapeDtypeStruct((_NC, _NPAD, _D2), jnp.float32),
        scratch_types=[
            pltpu.VMEM((n_ch, _SCH), jnp.int32),
            pltpu.VMEM((_SCH, _D2), jnp.float32),
            pltpu.VMEM((drc, _D2), jnp.float32),
            pltpu.VMEM_SHARED((_NPAD, _D2), jnp.float32),
            pltpu.SemaphoreType.DMA,
        ],
    )
    def k(idx3_hbm, zacc_hbm, ones_hbm, parts_hbm, idx_v, ones_v, buf_v,
          acc_sh, sem):
        c = lax.axis_index("c")
        s = lax.axis_index("s")
        wid = s * _NC + c
        r0 = s * rows_t
        pltpu.sync_copy(zacc_hbm, buf_v)
        for h in range(4):
            pltpu.sync_copy(buf_v, acc_sh.at[pl.ds(r0 + h * drc, drc)])
        pltpu.sync_copy(idx3_hbm.at[wid], idx_v)
        pltpu.sync_copy(ones_hbm, ones_v)
        plsc.subcore_barrier()

        def step(i, carry):
            pltpu.sync_copy(ones_v, acc_sh.at[idx_v.at[i]], add=True)
            return carry

        lax.fori_loop(0, n_ch, step, 0)
        plsc.subcore_barrier()
        for h in range(4):
            pltpu.sync_copy(acc_sh.at[pl.ds(r0 + h * drc, drc)], buf_v)
            pltpu.sync_copy(buf_v, parts_hbm.at[c, pl.ds(r0 + h * drc, drc)])

    return k(idx3, zeros_acc, ones_rows)


def _tw(lp):
    return lp["w"].T.astype(jnp.bfloat16), lp["b"][None, :]


def kernel(feat, efeat, edge_index, params):
    src = edge_index[0].astype(jnp.int32)
    dst = edge_index[1].astype(jnp.int32)
    idx_all = jnp.concatenate([src, dst])
    idx_all3 = idx_all.reshape(_NW, (2 * _N_EDGES // _NW) // _GCH, _GCH)
    dst3 = dst.reshape(_NW, (_N_EDGES // _NW) // _SCH, _SCH)
    ones_rows = jnp.ones((_SCH, _D2), jnp.float32)
    zeros_acc = jnp.zeros((_NPAD // _NS // 4, _D2), jnp.float32)

    cnts = None
    vout, eout = feat, efeat
    for n, bp in enumerate(params["blocks"]):
        fin, ein = vout, eout
        wd1, bd1 = _tw(bp["dense_node"][0])
        wd2, bd2 = _tw(bp["dense_node"][1])
        we1, be1 = _tw(bp["dense_edge"][0])
        we2, be2 = _tw(bp["dense_edge"][1])
        wvsk, bvsk = _tw(bp["edge_mlp0"]["vsk"])
        wvrk, bvrk = _tw(bp["edge_mlp0"]["vrk"])
        wek, bek = _tw(bp["edge_mlp0"]["ek"])
        wm1, bm1 = _tw(bp["edge_mlp"][0])
        wm2, bm2 = _tw(bp["edge_mlp"][1])
        wvi, bvi = _tw(bp["node_mlp0"]["vi"])
        wvie, bvie = _tw(bp["node_mlp0"]["vie"])
        wn1, bn1 = _tw(bp["node_mlp"][0])
        wn2, bn2 = _tw(bp["node_mlp"][1])

        hx, vi = _node_pre(fin, wd1, bd1, wd2, bd2, wvi, bvi)
        g = _sc_gather(hx, idx_all3)
        if n == 0:
            cnts = _sc_count(dst3, zeros_acc, ones_rows)
        b0 = bvsk + bvrk + bek
        e, eo = _edge(ein, g[:_N_EDGES], g[_N_EDGES:], we1, be1, we2, be2,
                      wvsk, wvrk, wek, b0, wm1, bm1, wm2, bm2,
                      residual_he=(n == 0))
        parts = _sc_scatter(e, dst3, zeros_acc)
        res = hx if n == 0 else fin
        v = _node_upd(parts[:, :_N_NODES], cnts[:, :_N_NODES], vi, res,
                      wvie, bvie, wn1, bn1, wn2, bn2)
        vout, eout = v, eo
    return vout, eout


# speedup vs baseline: 1.4308x; 1.0853x over previous
import functools

import jax
import jax.numpy as jnp
from jax import lax
from jax.experimental import pallas as pl
from jax.experimental.pallas import tpu as pltpu
from jax.experimental.pallas import tpu_sc as plsc

_N_NODES = 10000
_N_EDGES = 160000
_D1 = 256
_D2 = 128

_NPAD = 10240

_NC = 2
_NS = 16
_NW = _NC * _NS

_NB = 1000
_EB = 2000
_GCH = 80
_SCH = 40


def _relu(x):
    return jnp.maximum(x, 0.0)


def _dot(x, w):
    return jnp.dot(x.astype(jnp.bfloat16), w, preferred_element_type=jnp.float32)


def _node_pre(fin, wd1, bd1, wd2, bd2, wvi, bvi):
    f = fin.shape[1]
    grid = (_N_NODES // _NB,)

    def body(x_ref, wd1r, bd1r, wd2r, bd2r, wvir, bvir, hx_ref, vi_ref):
        x = x_ref[...]
        h = _relu(_dot(x, wd1r[...]) + bd1r[...])
        hx = _relu(_dot(h, wd2r[...]) + bd2r[...])
        hx_ref[...] = hx
        vi_ref[...] = _dot(hx, wvir[...]) + bvir[...]

    full = lambda a: pl.BlockSpec(a.shape, lambda i: (0,) * a.ndim)
    return pl.pallas_call(
        body,
        grid=grid,
        in_specs=[pl.BlockSpec((_NB, f), lambda i: (i, 0))]
        + [full(a) for a in (wd1, bd1, wd2, bd2, wvi, bvi)],
        out_specs=[
            pl.BlockSpec((_NB, _D2), lambda i: (i, 0)),
            pl.BlockSpec((_NB, _D1), lambda i: (i, 0)),
        ],
        out_shape=[
            jax.ShapeDtypeStruct((_N_NODES, _D2), jnp.float32),
            jax.ShapeDtypeStruct((_N_NODES, _D1), jnp.float32),
        ],
    )(fin, wd1, bd1, wd2, bd2, wvi, bvi)


def _edge(ein, gs, gd, wd1, bd1, wd2, bd2, wvsk, wvrk, wek, b0, wm1, bm1,
          wm2, bm2, residual_he):
    f = ein.shape[1]
    grid = (_N_EDGES // _EB,)

    def body(x_ref, gs_ref, gd_ref, wd1r, bd1r, wd2r, bd2r, wvskr, wvrkr,
             wekr, b0r, wm1r, bm1r, wm2r, bm2r, e_ref, eo_ref):
        x = x_ref[...]
        h = _relu(_dot(x, wd1r[...]) + bd1r[...])
        he = _relu(_dot(h, wd2r[...]) + bd2r[...])
        e0 = _relu(_dot(gs_ref[...], wvskr[...]) + _dot(gd_ref[...], wvrkr[...])
                   + _dot(he, wekr[...]) + b0r[...])
        h1 = _relu(_dot(e0, wm1r[...]) + bm1r[...])
        e = _relu(_dot(h1, wm2r[...]) + bm2r[...])
        e_ref[...] = e
        eo_ref[...] = e + (he if residual_he else x)

    full = lambda a: pl.BlockSpec(a.shape, lambda i: (0,) * a.ndim)
    return pl.pallas_call(
        body,
        grid=grid,
        in_specs=[
            pl.BlockSpec((_EB, f), lambda i: (i, 0)),
            pl.BlockSpec((_EB, _D2), lambda i: (i, 0)),
            pl.BlockSpec((_EB, _D2), lambda i: (i, 0)),
        ]
        + [full(a) for a in (wd1, bd1, wd2, bd2, wvsk, wvrk, wek, b0, wm1,
                             bm1, wm2, bm2)],
        out_specs=[
            pl.BlockSpec((_EB, _D2), lambda i: (i, 0)),
            pl.BlockSpec((_EB, _D2), lambda i: (i, 0)),
        ],
        out_shape=[
            jax.ShapeDtypeStruct((_N_EDGES, _D2), jnp.float32),
            jax.ShapeDtypeStruct((_N_EDGES, _D2), jnp.float32),
        ],
    )(ein, gs, gd, wd1, bd1, wd2, bd2, wvsk, wvrk, wek, b0, wm1, bm1,
      wm2, bm2)


def _node_upd(parts, cnts, vi, res, wvie, bvie, wn1, bn1, wn2, bn2):
    grid = (_N_NODES // _NB,)

    def body(p_ref, c_ref, vi_ref, r_ref, wvier, bvier, wn1r, bn1r, wn2r,
             bn2r, vo_ref):
        agg = p_ref[0] + p_ref[1]
        cnt = (c_ref[0] + c_ref[1])[:, 0:1]
        agg = agg / jnp.maximum(cnt, 1.0)
        vie = _dot(agg, wvier[...]) + bvier[...]
        v = _relu(vi_ref[...] + vie)
        v = _relu(_dot(v, wn1r[...]) + bn1r[...])
        v = _relu(_dot(v, wn2r[...]) + bn2r[...])
        vo_ref[...] = v + r_ref[...]

    full = lambda a: pl.BlockSpec(a.shape, lambda i: (0,) * a.ndim)
    return pl.pallas_call(
        body,
        grid=grid,
        in_specs=[
            pl.BlockSpec((2, _NB, _D2), lambda i: (0, i, 0)),
            pl.BlockSpec((2, _NB, _D2), lambda i: (0, i, 0)),
            pl.BlockSpec((_NB, _D1), lambda i: (i, 0)),
            pl.BlockSpec((_NB, _D2), lambda i: (i, 0)),
        ]
        + [full(a) for a in (wvie, bvie, wn1, bn1, wn2, bn2)],
        out_specs=pl.BlockSpec((_NB, _D2), lambda i: (i, 0)),
        out_shape=jax.ShapeDtypeStruct((_N_NODES, _D2), jnp.float32),
    )(parts, cnts, vi, res, wvie, bvie, wn1, bn1, wn2, bn2)


def _sc_gather(table, idx3):
    n_ch = idx3.shape[1]
    b = _NW * n_ch * _GCH
    d = table.shape[1]
    per_w = b // _NW
    mesh = plsc.VectorSubcoreMesh(core_axis_name="c", subcore_axis_name="s")
    n_pair = (n_ch - 1) // 2

    @functools.partial(
        pl.kernel,
        mesh=mesh,
        out_type=jax.ShapeDtypeStruct((b, d), jnp.float32),
        scratch_types=[
            pltpu.VMEM((n_ch, _GCH), jnp.int32),
            pltpu.VMEM((_GCH, d), jnp.float32),
            pltpu.VMEM((_GCH, d), jnp.float32),
            pltpu.SemaphoreType.DMA,
            pltpu.SemaphoreType.DMA,
        ],
    )
    def k(table_hbm, idx3_hbm, out_hbm, idx_v, rows_a, rows_b, sem_a, sem_b):
        wid = lax.axis_index("s") * _NC + lax.axis_index("c")
        base = wid * per_w
        pltpu.sync_copy(idx3_hbm.at[wid], idx_v)
        pltpu.async_copy(table_hbm.at[idx_v.at[0]], rows_a, sem_a)

        def wait_gather(rows, sem):
            pltpu.make_async_copy(table_hbm.at[idx_v.at[0]], rows, sem).wait()

        def step(g, carry):
            c0 = 2 * g
            pltpu.async_copy(table_hbm.at[idx_v.at[c0 + 1]], rows_b, sem_b)
            wait_gather(rows_a, sem_a)
            pltpu.sync_copy(rows_a, out_hbm.at[pl.ds(base + c0 * _GCH, _GCH)])
            pltpu.async_copy(table_hbm.at[idx_v.at[c0 + 2]], rows_a, sem_a)
            wait_gather(rows_b, sem_b)
            pltpu.sync_copy(rows_b,
                            out_hbm.at[pl.ds(base + (c0 + 1) * _GCH, _GCH)])
            return carry

        lax.fori_loop(0, n_pair, step, 0)
        wait_gather(rows_a, sem_a)
        pltpu.sync_copy(rows_a,
                        out_hbm.at[pl.ds(base + (n_ch - 1) * _GCH, _GCH)])

    return k(table, idx3)


def _sc_scatter(vals, idx3, zeros_acc):
    e = vals.shape[0]
    d = vals.shape[1]
    per_w = e // _NW
    n_ch = per_w // _SCH
    rows_t = _NPAD // _NS
    drc = rows_t // 4
    mesh = plsc.VectorSubcoreMesh(core_axis_name="c", subcore_axis_name="s")

    @functools.partial(
        pl.kernel,
        mesh=mesh,
        out_type=jax.ShapeDtypeStruct((_NC, _NPAD, d), jnp.float32),
        scratch_types=[
            pltpu.VMEM((n_ch, _SCH), jnp.int32),
            pltpu.VMEM((_SCH, d), jnp.float32),
            pltpu.VMEM((_SCH, d), jnp.float32),
            pltpu.VMEM((drc, d), jnp.float32),
            pltpu.VMEM_SHARED((_NPAD, d), jnp.float32),
            pltpu.SemaphoreType.DMA,
            pltpu.SemaphoreType.DMA,
        ],
    )
    def k(vals_hbm, idx3_hbm, zacc_hbm, parts_hbm, idx_v, rows_a, rows_b,
          buf_v, acc_sh, sem_a, sem_b):
        c = lax.axis_index("c")
        s = lax.axis_index("s")
        wid = s * _NC + c
        r0 = s * rows_t
        pltpu.sync_copy(zacc_hbm, buf_v)
        for h in range(4):
            pltpu.sync_copy(buf_v, acc_sh.at[pl.ds(r0 + h * drc, drc)])
        pltpu.sync_copy(idx3_hbm.at[wid], idx_v)
        plsc.subcore_barrier()
        base = wid * per_w
        n_pair = (n_ch - 1) // 2

        def load(i, rows, sem):
            pltpu.async_copy(vals_hbm.at[pl.ds(base + i * _SCH, _SCH)],
                             rows, sem)

        def wait_load(i, rows, sem):
            pltpu.make_async_copy(
                vals_hbm.at[pl.ds(base + i * _SCH, _SCH)], rows, sem).wait()

        load(0, rows_a, sem_a)

        def step(g, carry):
            c0 = 2 * g
            load(c0 + 1, rows_b, sem_b)
            wait_load(c0, rows_a, sem_a)
            pltpu.sync_copy(rows_a, acc_sh.at[idx_v.at[c0]], add=True)
            load(c0 + 2, rows_a, sem_a)
            wait_load(c0 + 1, rows_b, sem_b)
            pltpu.sync_copy(rows_b, acc_sh.at[idx_v.at[c0 + 1]], add=True)
            return carry

        lax.fori_loop(0, n_pair, step, 0)
        wait_load(n_ch - 1, rows_a, sem_a)
        pltpu.sync_copy(rows_a, acc_sh.at[idx_v.at[n_ch - 1]], add=True)
        plsc.subcore_barrier()
        for h in range(4):
            pltpu.sync_copy(acc_sh.at[pl.ds(r0 + h * drc, drc)], buf_v)
            pltpu.sync_copy(buf_v, parts_hbm.at[c, pl.ds(r0 + h * drc, drc)])

    return k(vals, idx3, zeros_acc)


def _sc_count(idx3, zeros_acc, ones_rows):
    n_ch = idx3.shape[1]
    rows_t = _NPAD // _NS
    drc = rows_t // 4
    mesh = plsc.VectorSubcoreMesh(core_axis_name="c", subcore_axis_name="s")

    @functools.partial(
        pl.kernel,
        mesh=mesh,
        out_type=jax.ShapeDtypeStruct((_NC, _NPAD, _D2), jnp.float32),
        scratch_types=[
            pltpu.VMEM((n_ch, _SCH), jnp.int32),
            pltpu.VMEM((_SCH, _D2), jnp.float32),
            pltpu.VMEM((drc, _D2), jnp.float32),
            pltpu.VMEM_SHARED((_NPAD, _D2), jnp.float32),
            pltpu.SemaphoreType.DMA,
        ],
    )
    def k(idx3_hbm, zacc_hbm, ones_hbm, parts_hbm, idx_v, ones_v, buf_v,
          acc_sh, sem):
        c = lax.axis_index("c")
        s = lax.axis_index("s")
        wid = s * _NC + c
        r0 = s * rows_t
        pltpu.sync_copy(zacc_hbm, buf_v)
        for h in range(4):
            pltpu.sync_copy(buf_v, acc_sh.at[pl.ds(r0 + h * drc, drc)])
        pltpu.sync_copy(idx3_hbm.at[wid], idx_v)
        pltpu.sync_copy(ones_hbm, ones_v)
        plsc.subcore_barrier()

        def step(i, carry):
            pltpu.sync_copy(ones_v, acc_sh.at[idx_v.at[i]], add=True)
            return carry

        lax.fori_loop(0, n_ch, step, 0)
        plsc.subcore_barrier()
        for h in range(4):
            pltpu.sync_copy(acc_sh.at[pl.ds(r0 + h * drc, drc)], buf_v)
            pltpu.sync_copy(buf_v, parts_hbm.at[c, pl.ds(r0 + h * drc, drc)])

    return k(idx3, zeros_acc, ones_rows)


def _tw(lp):
    return lp["w"].T.astype(jnp.bfloat16), lp["b"][None, :]


def kernel(feat, efeat, edge_index, params):
    src = edge_index[0].astype(jnp.int32)
    dst = edge_index[1].astype(jnp.int32)
    idx_all = jnp.concatenate([src, dst])
    idx_all3 = idx_all.reshape(_NW, (2 * _N_EDGES // _NW) // _GCH, _GCH)
    dst3 = dst.reshape(_NW, (_N_EDGES // _NW) // _SCH, _SCH)
    ones_rows = jnp.ones((_SCH, _D2), jnp.float32)
    zeros_acc = jnp.zeros((_NPAD // _NS // 4, _D2), jnp.float32)

    cnts = None
    vout, eout = feat, efeat
    for n, bp in enumerate(params["blocks"]):
        fin, ein = vout, eout
        wd1, bd1 = _tw(bp["dense_node"][0])
        wd2, bd2 = _tw(bp["dense_node"][1])
        we1, be1 = _tw(bp["dense_edge"][0])
        we2, be2 = _tw(bp["dense_edge"][1])
        wvsk, bvsk = _tw(bp["edge_mlp0"]["vsk"])
        wvrk, bvrk = _tw(bp["edge_mlp0"]["vrk"])
        wek, bek = _tw(bp["edge_mlp0"]["ek"])
        wm1, bm1 = _tw(bp["edge_mlp"][0])
        wm2, bm2 = _tw(bp["edge_mlp"][1])
        wvi, bvi = _tw(bp["node_mlp0"]["vi"])
        wvie, bvie = _tw(bp["node_mlp0"]["vie"])
        wn1, bn1 = _tw(bp["node_mlp"][0])
        wn2, bn2 = _tw(bp["node_mlp"][1])

        if n == 0:
            cnts = _sc_count(dst3, zeros_acc, ones_rows)
        hx, vi = _node_pre(fin, wd1, bd1, wd2, bd2, wvi, bvi)
        g = _sc_gather(hx, idx_all3)
        b0 = bvsk + bvrk + bek
        e, eo = _edge(ein, g[:_N_EDGES], g[_N_EDGES:], we1, be1, we2, be2,
                      wvsk, wvrk, wek, b0, wm1, bm1, wm2, bm2,
                      residual_he=(n == 0))
        parts = _sc_scatter(e, dst3, zeros_acc)
        res = hx if n == 0 else fin
        v = _node_upd(parts[:, :_N_NODES], cnts[:, :_N_NODES], vi, res,
                      wvie, bvie, wn1, bn1, wn2, bn2)
        vout, eout = v, eo
    return vout, eout


# speedup vs baseline: 1.4684x; 1.0263x over previous
import functools

import jax
import jax.numpy as jnp
from jax import lax
from jax.experimental import pallas as pl
from jax.experimental.pallas import tpu as pltpu
from jax.experimental.pallas import tpu_sc as plsc

_N_NODES = 10000
_N_EDGES = 160000
_D1 = 256
_D2 = 128

_NPAD = 10240

_NC = 2
_NS = 16
_NW = _NC * _NS

_NB = 1000
_EB = 4000
_GCH = 80
_SCH = 40


def _relu(x):
    return jnp.maximum(x, 0.0)


def _dot(x, w):
    return jnp.dot(x.astype(jnp.bfloat16), w, preferred_element_type=jnp.float32)


def _node_pre(fin, wd1, bd1, wd2, bd2, wvi, bvi):
    f = fin.shape[1]
    grid = (_N_NODES // _NB,)

    def body(x_ref, wd1r, bd1r, wd2r, bd2r, wvir, bvir, hx_ref, vi_ref):
        x = x_ref[...]
        h = _relu(_dot(x, wd1r[...]) + bd1r[...])
        hx = _relu(_dot(h, wd2r[...]) + bd2r[...])
        hx_ref[...] = hx
        vi_ref[...] = _dot(hx, wvir[...]) + bvir[...]

    full = lambda a: pl.BlockSpec(a.shape, lambda i: (0,) * a.ndim)
    return pl.pallas_call(
        body,
        grid=grid,
        in_specs=[pl.BlockSpec((_NB, f), lambda i: (i, 0))]
        + [full(a) for a in (wd1, bd1, wd2, bd2, wvi, bvi)],
        out_specs=[
            pl.BlockSpec((_NB, _D2), lambda i: (i, 0)),
            pl.BlockSpec((_NB, _D1), lambda i: (i, 0)),
        ],
        out_shape=[
            jax.ShapeDtypeStruct((_N_NODES, _D2), jnp.float32),
            jax.ShapeDtypeStruct((_N_NODES, _D1), jnp.float32),
        ],
    )(fin, wd1, bd1, wd2, bd2, wvi, bvi)


def _edge(ein, gs, gd, wd1, bd1, wd2, bd2, wvsk, wvrk, wek, b0, wm1, bm1,
          wm2, bm2, residual_he):
    f = ein.shape[1]
    grid = (_N_EDGES // _EB,)

    def body(x_ref, gs_ref, gd_ref, wd1r, bd1r, wd2r, bd2r, wvskr, wvrkr,
             wekr, b0r, wm1r, bm1r, wm2r, bm2r, e_ref, eo_ref):
        x = x_ref[...]
        h = _relu(_dot(x, wd1r[...]) + bd1r[...])
        he = _relu(_dot(h, wd2r[...]) + bd2r[...])
        e0 = _relu(_dot(gs_ref[...], wvskr[...]) + _dot(gd_ref[...], wvrkr[...])
                   + _dot(he, wekr[...]) + b0r[...])
        h1 = _relu(_dot(e0, wm1r[...]) + bm1r[...])
        e = _relu(_dot(h1, wm2r[...]) + bm2r[...])
        e_ref[...] = e
        eo_ref[...] = e + (he if residual_he else x)

    full = lambda a: pl.BlockSpec(a.shape, lambda i: (0,) * a.ndim)
    return pl.pallas_call(
        body,
        grid=grid,
        in_specs=[
            pl.BlockSpec((_EB, f), lambda i: (i, 0)),
            pl.BlockSpec((_EB, _D2), lambda i: (i, 0)),
            pl.BlockSpec((_EB, _D2), lambda i: (i, 0)),
        ]
        + [full(a) for a in (wd1, bd1, wd2, bd2, wvsk, wvrk, wek, b0, wm1,
                             bm1, wm2, bm2)],
        out_specs=[
            pl.BlockSpec((_EB, _D2), lambda i: (i, 0)),
            pl.BlockSpec((_EB, _D2), lambda i: (i, 0)),
        ],
        out_shape=[
            jax.ShapeDtypeStruct((_N_EDGES, _D2), jnp.float32),
            jax.ShapeDtypeStruct((_N_EDGES, _D2), jnp.float32),
        ],
    )(ein, gs, gd, wd1, bd1, wd2, bd2, wvsk, wvrk, wek, b0, wm1, bm1,
      wm2, bm2)


def _node_upd(parts, cnts, vi, res, wvie, bvie, wn1, bn1, wn2, bn2):
    grid = (_N_NODES // _NB,)

    def body(p_ref, c_ref, vi_ref, r_ref, wvier, bvier, wn1r, bn1r, wn2r,
             bn2r, vo_ref):
        agg = p_ref[0] + p_ref[1]
        cnt = (c_ref[0] + c_ref[1])[:, 0:1]
        agg = agg / jnp.maximum(cnt, 1.0)
        vie = _dot(agg, wvier[...]) + bvier[...]
        v = _relu(vi_ref[...] + vie)
        v = _relu(_dot(v, wn1r[...]) + bn1r[...])
        v = _relu(_dot(v, wn2r[...]) + bn2r[...])
        vo_ref[...] = v + r_ref[...]

    full = lambda a: pl.BlockSpec(a.shape, lambda i: (0,) * a.ndim)
    return pl.pallas_call(
        body,
        grid=grid,
        in_specs=[
            pl.BlockSpec((2, _NB, _D2), lambda i: (0, i, 0)),
            pl.BlockSpec((2, _NB, _D2), lambda i: (0, i, 0)),
            pl.BlockSpec((_NB, _D1), lambda i: (i, 0)),
            pl.BlockSpec((_NB, _D2), lambda i: (i, 0)),
        ]
        + [full(a) for a in (wvie, bvie, wn1, bn1, wn2, bn2)],
        out_specs=pl.BlockSpec((_NB, _D2), lambda i: (i, 0)),
        out_shape=jax.ShapeDtypeStruct((_N_NODES, _D2), jnp.float32),
    )(parts, cnts, vi, res, wvie, bvie, wn1, bn1, wn2, bn2)


def _sc_gather(table, idx3):
    n_ch = idx3.shape[1]
    b = _NW * n_ch * _GCH
    d = table.shape[1]
    per_w = b // _NW
    mesh = plsc.VectorSubcoreMesh(core_axis_name="c", subcore_axis_name="s")
    n_pair = (n_ch - 1) // 2

    @functools.partial(
        pl.kernel,
        mesh=mesh,
        out_type=jax.ShapeDtypeStruct((b, d), jnp.float32),
        scratch_types=[
            pltpu.VMEM((n_ch, _GCH), jnp.int32),
            pltpu.VMEM((_GCH, d), jnp.float32),
            pltpu.VMEM((_GCH, d), jnp.float32),
            pltpu.SemaphoreType.DMA,
            pltpu.SemaphoreType.DMA,
        ],
    )
    def k(table_hbm, idx3_hbm, out_hbm, idx_v, rows_a, rows_b, sem_a, sem_b):
        wid = lax.axis_index("s") * _NC + lax.axis_index("c")
        base = wid * per_w
        pltpu.sync_copy(idx3_hbm.at[wid], idx_v)
        pltpu.async_copy(table_hbm.at[idx_v.at[0]], rows_a, sem_a)

        def wait_gather(rows, sem):
            pltpu.make_async_copy(table_hbm.at[idx_v.at[0]], rows, sem).wait()

        def step(g, carry):
            c0 = 2 * g
            pltpu.async_copy(table_hbm.at[idx_v.at[c0 + 1]], rows_b, sem_b)
            wait_gather(rows_a, sem_a)
            pltpu.sync_copy(rows_a, out_hbm.at[pl.ds(base + c0 * _GCH, _GCH)])
            pltpu.async_copy(table_hbm.at[idx_v.at[c0 + 2]], rows_a, sem_a)
            wait_gather(rows_b, sem_b)
            pltpu.sync_copy(rows_b,
                            out_hbm.at[pl.ds(base + (c0 + 1) * _GCH, _GCH)])
            return carry

        lax.fori_loop(0, n_pair, step, 0)
        wait_gather(rows_a, sem_a)
        pltpu.sync_copy(rows_a,
                        out_hbm.at[pl.ds(base + (n_ch - 1) * _GCH, _GCH)])

    return k(table, idx3)


def _sc_scatter(vals, idx3, zeros_acc):
    e = vals.shape[0]
    d = vals.shape[1]
    per_w = e // _NW
    n_ch = per_w // _SCH
    rows_t = _NPAD // _NS
    drc = rows_t // 4
    mesh = plsc.VectorSubcoreMesh(core_axis_name="c", subcore_axis_name="s")

    @functools.partial(
        pl.kernel,
        mesh=mesh,
        out_type=jax.ShapeDtypeStruct((_NC, _NPAD, d), jnp.float32),
        scratch_types=[
            pltpu.VMEM((n_ch, _SCH), jnp.int32),
            pltpu.VMEM((_SCH, d), jnp.float32),
            pltpu.VMEM((_SCH, d), jnp.float32),
            pltpu.VMEM((drc, d), jnp.float32),
            pltpu.VMEM_SHARED((_NPAD, d), jnp.float32),
            pltpu.SemaphoreType.DMA,
            pltpu.SemaphoreType.DMA,
        ],
    )
    def k(vals_hbm, idx3_hbm, zacc_hbm, parts_hbm, idx_v, rows_a, rows_b,
          buf_v, acc_sh, sem_a, sem_b):
        c = lax.axis_index("c")
        s = lax.axis_index("s")
        wid = s * _NC + c
        r0 = s * rows_t
        pltpu.sync_copy(zacc_hbm, buf_v)
        for h in range(4):
            pltpu.sync_copy(buf_v, acc_sh.at[pl.ds(r0 + h * drc, drc)])
        pltpu.sync_copy(idx3_hbm.at[wid], idx_v)
        plsc.subcore_barrier()
        base = wid * per_w
        n_pair = (n_ch - 1) // 2

        def load(i, rows, sem):
            pltpu.async_copy(vals_hbm.at[pl.ds(base + i * _SCH, _SCH)],
                             rows, sem)

        def wait_load(i, rows, sem):
            pltpu.make_async_copy(
                vals_hbm.at[pl.ds(base + i * _SCH, _SCH)], rows, sem).wait()

        load(0, rows_a, sem_a)

        def step(g, carry):
            c0 = 2 * g
            load(c0 + 1, rows_b, sem_b)
            wait_load(c0, rows_a, sem_a)
            pltpu.sync_copy(rows_a, acc_sh.at[idx_v.at[c0]], add=True)
            load(c0 + 2, rows_a, sem_a)
            wait_load(c0 + 1, rows_b, sem_b)
            pltpu.sync_copy(rows_b, acc_sh.at[idx_v.at[c0 + 1]], add=True)
            return carry

        lax.fori_loop(0, n_pair, step, 0)
        wait_load(n_ch - 1, rows_a, sem_a)
        pltpu.sync_copy(rows_a, acc_sh.at[idx_v.at[n_ch - 1]], add=True)
        plsc.subcore_barrier()
        for h in range(4):
            pltpu.sync_copy(acc_sh.at[pl.ds(r0 + h * drc, drc)], buf_v)
            pltpu.sync_copy(buf_v, parts_hbm.at[c, pl.ds(r0 + h * drc, drc)])

    return k(vals, idx3, zeros_acc)


def _sc_count(idx3, zeros_acc, ones_rows):
    n_ch = idx3.shape[1]
    rows_t = _NPAD // _NS
    drc = rows_t // 4
    mesh = plsc.VectorSubcoreMesh(core_axis_name="c", subcore_axis_name="s")

    @functools.partial(
        pl.kernel,
        mesh=mesh,
        out_type=jax.ShapeDtypeStruct((_NC, _NPAD, _D2), jnp.float32),
        scratch_types=[
            pltpu.VMEM((n_ch, _SCH), jnp.int32),
            pltpu.VMEM((_SCH, _D2), jnp.float32),
            pltpu.VMEM((drc, _D2), jnp.float32),
            pltpu.VMEM_SHARED((_NPAD, _D2), jnp.float32),
            pltpu.SemaphoreType.DMA,
        ],
    )
    def k(idx3_hbm, zacc_hbm, ones_hbm, parts_hbm, idx_v, ones_v, buf_v,
          acc_sh, sem):
        c = lax.axis_index("c")
        s = lax.axis_index("s")
        wid = s * _NC + c
        r0 = s * rows_t
        pltpu.sync_copy(zacc_hbm, buf_v)
        for h in range(4):
            pltpu.sync_copy(buf_v, acc_sh.at[pl.ds(r0 + h * drc, drc)])
        pltpu.sync_copy(idx3_hbm.at[wid], idx_v)
        pltpu.sync_copy(ones_hbm, ones_v)
        plsc.subcore_barrier()

        def step(i, carry):
            pltpu.sync_copy(ones_v, acc_sh.at[idx_v.at[i]], add=True)
            return carry

        lax.fori_loop(0, n_ch, step, 0)
        plsc.subcore_barrier()
        for h in range(4):
            pltpu.sync_copy(acc_sh.at[pl.ds(r0 + h * drc, drc)], buf_v)
            pltpu.sync_copy(buf_v, parts_hbm.at[c, pl.ds(r0 + h * drc, drc)])

    return k(idx3, zeros_acc, ones_rows)


def _tw(lp):
    return lp["w"].T.astype(jnp.bfloat16), lp["b"][None, :]


def kernel(feat, efeat, edge_index, params):
    src = edge_index[0].astype(jnp.int32)
    dst = edge_index[1].astype(jnp.int32)
    idx_all = jnp.concatenate([src, dst])
    idx_all3 = idx_all.reshape(_NW, (2 * _N_EDGES // _NW) // _GCH, _GCH)
    dst3 = dst.reshape(_NW, (_N_EDGES // _NW) // _SCH, _SCH)
    ones_rows = jnp.ones((_SCH, _D2), jnp.float32)
    zeros_acc = jnp.zeros((_NPAD // _NS // 4, _D2), jnp.float32)

    cnts = None
    vout, eout = feat, efeat
    for n, bp in enumerate(params["blocks"]):
        fin, ein = vout, eout
        wd1, bd1 = _tw(bp["dense_node"][0])
        wd2, bd2 = _tw(bp["dense_node"][1])
        we1, be1 = _tw(bp["dense_edge"][0])
        we2, be2 = _tw(bp["dense_edge"][1])
        wvsk, bvsk = _tw(bp["edge_mlp0"]["vsk"])
        wvrk, bvrk = _tw(bp["edge_mlp0"]["vrk"])
        wek, bek = _tw(bp["edge_mlp0"]["ek"])
        wm1, bm1 = _tw(bp["edge_mlp"][0])
        wm2, bm2 = _tw(bp["edge_mlp"][1])
        wvi, bvi = _tw(bp["node_mlp0"]["vi"])
        wvie, bvie = _tw(bp["node_mlp0"]["vie"])
        wn1, bn1 = _tw(bp["node_mlp"][0])
        wn2, bn2 = _tw(bp["node_mlp"][1])

        if n == 0:
            cnts = _sc_count(dst3, zeros_acc, ones_rows)
        hx, vi = _node_pre(fin, wd1, bd1, wd2, bd2, wvi, bvi)
        g = _sc_gather(hx, idx_all3)
        b0 = bvsk + bvrk + bek
        e, eo = _edge(ein, g[:_N_EDGES], g[_N_EDGES:], we1, be1, we2, be2,
                      wvsk, wvrk, wek, b0, wm1, bm1, wm2, bm2,
                      residual_he=(n == 0))
        parts = _sc_scatter(e, dst3, zeros_acc)
        res = hx if n == 0 else fin
        v = _node_upd(parts[:, :_N_NODES], cnts[:, :_N_NODES], vi, res,
                      wvie, bvie, wn1, bn1, wn2, bn2)
        vout, eout = v, eo
    return vout, eout
